# Initial kernel scaffold; baseline (speedup 1.0000x reference)
import jax
import jax.numpy as jnp
from jax.experimental import pallas as pl


def kernel(x, a, b, alpha, id_gain, bias):
    raise NotImplementedError("write your pallas kernel here")


# speedup vs baseline: 1420.2318x; 1420.2318x over previous
import jax
import jax.numpy as jnp
from jax import lax
from jax.experimental import pallas as pl
from jax.experimental.pallas import tpu as pltpu

_CLAMP = 1.5


def _spline_kernel(x_ref, w_ref, o_ref):
    w = w_ref[...]
    A0 = w[:, 0:36]
    A1 = w[:, 1:37]
    A2 = w[:, 2:38]
    A3 = w[:, 3:39]
    p0 = (A0 + 4.0 * A1 + A2) * (1.0 / 6.0)
    p1 = (A2 - A0) * 0.5
    p2 = (A0 + A2) * 0.5 - A1
    p3 = (A3 - A0 + 3.0 * (A1 - A2)) * (1.0 / 6.0)

    a = w[:, 40:41]
    b = w[:, 41:42]
    g = w[:, 42:43]
    bias = w[:, 43:44]

    x = x_ref[...]
    kk = 15.5
    lim = _CLAMP * kk
    y = lax.clamp(-lim, x * (a * kk) + b * kk, lim)
    u = y + kk
    fi = jnp.floor(u)
    t = u - fi
    m = jnp.clip(fi.astype(jnp.int32) + 2, 0, 35)
    q0 = jnp.take_along_axis(p0, m, axis=1)
    q1 = jnp.take_along_axis(p1, m, axis=1)
    q2 = jnp.take_along_axis(p2, m, axis=1)
    q3 = jnp.take_along_axis(p3, m, axis=1)
    s = ((q3 * t + q2) * t + q1) * t + q0
    o_ref[...] = g * x + s + bias


def kernel(x, a, b, alpha, id_gain, bias):
    B, C, H, W = x.shape
    K = alpha.shape[-1]
    HW = H * W
    x2 = x.reshape(B * C, HW)

    pad_idx = jnp.clip(jnp.arange(40) - 3, 0, K - 1)
    alpha_pad = alpha[:, pad_idx]
    w = jnp.concatenate(
        [alpha_pad, a[:, None], b[:, None], id_gain[:, None], bias[:, None]],
        axis=1,
    )

    LB = HW // 2
    grid = (B, HW // LB)
    out = pl.pallas_call(
        _spline_kernel,
        grid=grid,
        in_specs=[
            pl.BlockSpec((C, LB), lambda i, j: (i, j)),
            pl.BlockSpec((C, 44), lambda i, j: (0, 0)),
        ],
        out_specs=pl.BlockSpec((C, LB), lambda i, j: (i, j)),
        out_shape=jax.ShapeDtypeStruct((B * C, HW), jnp.float32),
        compiler_params=pltpu.CompilerParams(
            dimension_semantics=("parallel", "arbitrary"),
        ),
    )(x2, w)
    return out.reshape(B, C, H, W)
